# baseline (speedup 1.0000x reference)
import dataclasses

import jax
import jax.numpy as jnp
from jax import lax
from jax.experimental import pallas as pl
from jax.experimental.pallas import tpu as pltpu
from jax.experimental.pallas import tpu_sc as plsc

_START_VAL = 60.0
_END_VAL = 61.0

_B, _L = 16384, 200
_LOUT = _L + 2
_NW = 32
_CH = 64
_NSLICE = (_LOUT + 15) // 16
_SPLIT = 8192
_BR = 512


def _sc_body(codes_hbm, table_hbm, out_hbm, table_v):
    pltpu.sync_copy(table_hbm, table_v)
    lane = lax.iota(jnp.int32, 16)

    cidxs = []
    for k in range(_NSLICE):
        c0 = 16 * k
        cidx = lane + (c0 - 1)
        if k == 0:
            cidx = jnp.maximum(cidx, 0)
        if 16 * (k + 1) > _L:
            cidx = jnp.minimum(cidx, _L - 1)
        cidxs.append(cidx)
    lastcol = lane + 16 * (_NSLICE - 1)
    lastcol_st = jnp.minimum(lastcol, _LOUT - 1)
    lastmask = lastcol <= _LOUT - 1
    startmask = lane == 0

    def _compute(codes_v, out_v):
        @plsc.parallel_loop(0, _CH, unroll=4)
        def _row(r):
            rsplat = jnp.broadcast_to(r, (16,))
            for k in range(_NSLICE):
                c0 = 16 * k
                codes16 = plsc.load_gather(codes_v, [rsplat, cidxs[k]])
                tok = plsc.load_gather(table_v, [codes16])
                if k == 0:
                    tok = jnp.where(startmask, jnp.float32(_START_VAL), tok)
                if k == _NSLICE - 1:
                    tok = jnp.where(lastcol == _LOUT - 1,
                                    jnp.float32(_END_VAL), tok)
                    plsc.store_scatter(out_v, [rsplat, lastcol_st], tok,
                                       mask=lastmask)
                else:
                    out_v[r, pl.ds(c0, 16)] = tok

    pltpu.emit_pipeline(
        _compute,
        grid=(_SPLIT // _CH,),
        in_specs=[pl.BlockSpec((_CH, _L), lambda i: (i, 0))],
        out_specs=[pl.BlockSpec((_CH, _LOUT), lambda i: (i, 0))],
        core_axis_name=("c", "s"),
        dimension_semantics=(pltpu.PARALLEL,),
    )(codes_hbm, out_hbm)


def _sc_part(char_codes, lookup_table):
    mesh = plsc.VectorSubcoreMesh(core_axis_name="c", subcore_axis_name="s")
    cp = pltpu.CompilerParams()
    if "needs_layout_passes" in pltpu.CompilerParams.__dataclass_fields__:
        cp = dataclasses.replace(cp, needs_layout_passes=False)
    sc = pl.kernel(
        _sc_body,
        out_type=jax.ShapeDtypeStruct((_SPLIT, _LOUT), jnp.float32),
        mesh=mesh,
        scratch_types=[pltpu.VMEM((128,), jnp.float32)],
        compiler_params=cp,
    )
    return sc(char_codes, lookup_table)


def _tc_body(codes_ref, table_ref, out_ref):
    codes = codes_ref[...]
    table = table_ref[...]
    br = codes.shape[0]
    tb = jnp.broadcast_to(table, (br, 128))
    vals = jnp.take_along_axis(tb, codes, axis=1, mode="promise_in_bounds")
    start = jnp.full((br, 1), _START_VAL, jnp.float32)
    end = jnp.full((br, 1), _END_VAL, jnp.float32)
    out_ref[...] = jnp.concatenate([start, vals, end], axis=1)


def _tc_part(char_codes, lookup_table):
    nrows = _B - _SPLIT
    off = _SPLIT // _BR
    table2d = lookup_table.reshape(1, 128)
    return pl.pallas_call(
        _tc_body,
        grid=(nrows // _BR,),
        in_specs=[
            pl.BlockSpec((_BR, _L), lambda i: (i + off, 0)),
            pl.BlockSpec((1, 128), lambda i: (0, 0)),
        ],
        out_specs=pl.BlockSpec((_BR, _LOUT), lambda i: (i, 0)),
        out_shape=jax.ShapeDtypeStruct((nrows, _LOUT), jnp.float32),
    )(char_codes, table2d)


def kernel(char_codes, lookup_table):
    out_sc = _sc_part(char_codes, lookup_table)
    out_tc = _tc_part(char_codes, lookup_table)
    return jnp.concatenate([out_sc, out_tc], axis=0)


# speedup vs baseline: 1.1771x; 1.1771x over previous
import dataclasses
import functools

import jax
import jax.numpy as jnp
from jax import lax
from jax.experimental import pallas as pl
from jax.experimental.pallas import tpu as pltpu
from jax.experimental.pallas import tpu_sc as plsc

_START_VAL = 60.0
_END_VAL = 61.0

_B, _L = 16384, 200
_LOUT = _L + 2
_NW = 32
_ROWS_PER_W = _B // _NW
_CH = 64
_N_CHUNKS = _ROWS_PER_W // _CH
_NSLICE = (_LOUT + 15) // 16


def _sc_body(codes_hbm, table_hbm, out_hbm, table_v):
    pltpu.sync_copy(table_hbm, table_v)
    lane = lax.iota(jnp.int32, 16)

    cidxs = []
    for k in range(_NSLICE):
        c0 = 16 * k
        cidx = lane + (c0 - 1)
        if k == 0:
            cidx = jnp.maximum(cidx, 0)
        if 16 * (k + 1) > _L:
            cidx = jnp.minimum(cidx, _L - 1)
        cidxs.append(cidx)
    lastcol = lane + 16 * (_NSLICE - 1)
    lastcol_st = jnp.minimum(lastcol, _LOUT - 1)
    lastmask = lastcol <= _LOUT - 1

    def _compute(codes_v, out_v):
        @plsc.parallel_loop(0, _CH, unroll=6)
        def _row(r):
            rsplat = jnp.broadcast_to(r, (16,))
            for k in range(_NSLICE):
                c0 = 16 * k
                codes16 = plsc.load_gather(codes_v, [rsplat, cidxs[k]])
                tok = plsc.load_gather(table_v, [codes16])
                if k == 0:
                    tok = jnp.where(lane == 0, jnp.float32(_START_VAL), tok)
                if k == _NSLICE - 1:
                    tok = jnp.where(lastcol == _LOUT - 1, jnp.float32(_END_VAL),
                                    tok)
                    plsc.store_scatter(out_v, [rsplat, lastcol_st], tok,
                                       mask=lastmask)
                else:
                    out_v[r, pl.ds(c0, 16)] = tok

    pltpu.emit_pipeline(
        _compute,
        grid=(_B // _CH,),
        in_specs=[pl.BlockSpec((_CH, _L), lambda i: (i, 0))],
        out_specs=[pl.BlockSpec((_CH, _LOUT), lambda i: (i, 0))],
        core_axis_name=("c", "s"),
        dimension_semantics=(pltpu.PARALLEL,),
    )(codes_hbm, out_hbm)


def kernel(char_codes, lookup_table):
    B, L = char_codes.shape
    mesh = plsc.VectorSubcoreMesh(core_axis_name="c", subcore_axis_name="s")
    cp = pltpu.CompilerParams()
    if "needs_layout_passes" in pltpu.CompilerParams.__dataclass_fields__:
        cp = dataclasses.replace(cp, needs_layout_passes=False)
    sc = pl.kernel(
        _sc_body,
        out_type=jax.ShapeDtypeStruct((B, L + 2), jnp.float32),
        mesh=mesh,
        scratch_types=[
            pltpu.VMEM((128,), jnp.float32),
        ],
        compiler_params=cp,
    )
    return sc(char_codes, lookup_table)


# speedup vs baseline: 1.2086x; 1.0268x over previous
import dataclasses
import functools

import jax
import jax.numpy as jnp
from jax import lax
from jax.experimental import pallas as pl
from jax.experimental.pallas import tpu as pltpu
from jax.experimental.pallas import tpu_sc as plsc

_START_VAL = 60.0
_END_VAL = 61.0

_B, _L = 16384, 200
_LOUT = _L + 2
_NW = 32
_ROWS_PER_W = _B // _NW
_CH = 64
_N_CHUNKS = _ROWS_PER_W // _CH
_NSLICE = (_LOUT + 15) // 16


def _sc_body(codes_hbm, table_hbm, out_hbm, table_v):
    pltpu.sync_copy(table_hbm, table_v)
    lane = lax.iota(jnp.int32, 16)

    cidxs = []
    for k in range(_NSLICE):
        c0 = 16 * k
        cidx = lane + (c0 - 1)
        if k == 0:
            cidx = jnp.maximum(cidx, 0)
        if 16 * (k + 1) > _L:
            cidx = jnp.minimum(cidx, _L - 1)
        cidxs.append(cidx)
    lastcol = lane + 16 * (_NSLICE - 1)
    lastcol_st = jnp.minimum(lastcol, _LOUT - 1)
    lastmask = lastcol <= _LOUT - 1

    def _compute(codes_v, out_v):
        @plsc.parallel_loop(0, _CH, unroll=4)
        def _row(r):
            rsplat = jnp.broadcast_to(r, (16,))
            for k in range(_NSLICE):
                c0 = 16 * k
                codes16 = plsc.load_gather(codes_v, [rsplat, cidxs[k]])
                tok = plsc.load_gather(table_v, [codes16])
                if k == 0:
                    tok = jnp.where(lane == 0, jnp.float32(_START_VAL), tok)
                if k == _NSLICE - 1:
                    tok = jnp.where(lastcol == _LOUT - 1, jnp.float32(_END_VAL),
                                    tok)
                    plsc.store_scatter(out_v, [rsplat, lastcol_st], tok,
                                       mask=lastmask)
                else:
                    out_v[r, pl.ds(c0, 16)] = tok

    pltpu.emit_pipeline(
        _compute,
        grid=(_B // _CH,),
        in_specs=[pl.BlockSpec((_CH, _L), lambda i: (i, 0))],
        out_specs=[pl.BlockSpec((_CH, _LOUT), lambda i: (i, 0))],
        core_axis_name=("c", "s"),
        dimension_semantics=(pltpu.PARALLEL,),
        trace_scopes=False,
    )(codes_hbm, out_hbm)


def kernel(char_codes, lookup_table):
    B, L = char_codes.shape
    mesh = plsc.VectorSubcoreMesh(core_axis_name="c", subcore_axis_name="s")
    cp = pltpu.CompilerParams()
    if "needs_layout_passes" in pltpu.CompilerParams.__dataclass_fields__:
        cp = dataclasses.replace(cp, needs_layout_passes=False)
    sc = pl.kernel(
        _sc_body,
        out_type=jax.ShapeDtypeStruct((B, L + 2), jnp.float32),
        mesh=mesh,
        scratch_types=[
            pltpu.VMEM((128,), jnp.float32),
        ],
        compiler_params=cp,
    )
    return sc(char_codes, lookup_table)


# speedup vs baseline: 1.2089x; 1.0002x over previous
import dataclasses

import jax
import jax.numpy as jnp
from jax import lax
from jax.experimental import pallas as pl
from jax.experimental.pallas import tpu as pltpu
from jax.experimental.pallas import tpu_sc as plsc

_START_VAL = 60.0
_END_VAL = 61.0

_B, _L = 16384, 200
_LOUT = _L + 2
_CH = 64
_NSLICE = (_LOUT + 15) // 16


def _sc_body(codes_hbm, table_hbm, out_hbm, table_v):
    pltpu.sync_copy(table_hbm, table_v)
    lane = lax.iota(jnp.int32, 16)

    cidxs = []
    for k in range(_NSLICE):
        cidx = lane + (16 * k - 1)
        if k == 0:
            cidx = jnp.maximum(cidx, 0)
        if 16 * (k + 1) > _L:
            cidx = jnp.minimum(cidx, _L - 1)
        cidxs.append(cidx)
    lastcol = lane + 16 * (_NSLICE - 1)
    lastcol_st = jnp.minimum(lastcol, _LOUT - 1)
    lastmask = lastcol <= _LOUT - 1
    startmask = lane == 0

    def _compute(codes_v, out_v):
        @plsc.parallel_loop(0, _CH, unroll=4)
        def _row(r):
            rsplat = jnp.broadcast_to(r, (16,))
            for k in range(_NSLICE):
                codes16 = plsc.load_gather(codes_v, [rsplat, cidxs[k]])
                tok = plsc.load_gather(table_v, [codes16])
                if k == 0:
                    tok = jnp.where(startmask, jnp.float32(_START_VAL), tok)
                if k == _NSLICE - 1:
                    tok = jnp.where(lastcol == _LOUT - 1,
                                    jnp.float32(_END_VAL), tok)
                    plsc.store_scatter(out_v, [rsplat, lastcol_st], tok,
                                       mask=lastmask)
                else:
                    out_v[r, pl.ds(16 * k, 16)] = tok

    pltpu.emit_pipeline(
        _compute,
        grid=(_B // _CH,),
        in_specs=[pl.BlockSpec((_CH, _L), lambda i: (i, 0))],
        out_specs=[pl.BlockSpec((_CH, _LOUT), lambda i: (i, 0))],
        core_axis_name=("c", "s"),
        dimension_semantics=(pltpu.PARALLEL,),
        trace_scopes=False,
    )(codes_hbm, out_hbm)


def kernel(char_codes, lookup_table):
    B, L = char_codes.shape
    mesh = plsc.VectorSubcoreMesh(core_axis_name="c", subcore_axis_name="s")
    cp = pltpu.CompilerParams()
    if "needs_layout_passes" in pltpu.CompilerParams.__dataclass_fields__:
        cp = dataclasses.replace(cp, needs_layout_passes=False)
    sc = pl.kernel(
        _sc_body,
        out_type=jax.ShapeDtypeStruct((B, L + 2), jnp.float32),
        mesh=mesh,
        scratch_types=[pltpu.VMEM((128,), jnp.float32)],
        compiler_params=cp,
    )
    return sc(char_codes, lookup_table)
